# baseline (speedup 1.0000x reference)
import functools

import jax
import jax.numpy as jnp
from jax import lax
from jax.experimental import pallas as pl
from jax.experimental.pallas import tpu as pltpu
from jax.experimental.pallas import tpu_sc as plsc

N_FIELDS = 26
VOCAB = 100000
EMB = 32
BATCH = 16384

_NW = 32
_BW = BATCH // _NW
_BB = 64
_NB = _BW // _BB
_OUTW = N_FIELDS * EMB


def _body(x, tab, out, xv, idx2, rows2, asm, sem0, sem1):
    wid = lax.axis_index("s") * 2 + lax.axis_index("c")
    b0 = wid * _BW
    pltpu.sync_copy(x.at[pl.ds(b0, _BW)], xv)

    lane = lax.iota(jnp.int32, 16)
    sems = (sem0, sem1)

    def fill_idx(bb, f, buf):
        col = jnp.full((16,), f, jnp.int32)
        for s in range(_BB // 16):
            row = bb * _BB + s * 16 + lane
            idx2[buf, pl.ds(s * 16, 16)] = plsc.load_gather(xv, [row, col])

    def fire(bb, f, buf):
        fill_idx(bb, f, buf)
        return pltpu.async_copy(
            tab.at[f].at[idx2.at[buf]], rows2.at[buf], sems[buf]
        )

    def block(bb, _):
        fire(bb, 0, 0)
        for f in range(N_FIELDS):
            buf = f % 2
            if f + 1 < N_FIELDS:
                fire(bb, f + 1, 1 - buf)
            pltpu.make_async_copy(
                tab.at[f].at[idx2.at[buf]], rows2.at[buf], sems[buf]
            ).wait()

            def interleave(t, _):
                for k in range(8):
                    for c in (0, 16):
                        asm[t * 8 + k, pl.ds(f * EMB + c, 16)] = (
                            rows2[buf, t * 8 + k, pl.ds(c, 16)]
                        )
                return ()

            lax.fori_loop(0, _BB // 8, interleave, ())
        pltpu.sync_copy(asm, out.at[pl.ds(b0 + bb * _BB, _BB)])
        return ()

    lax.fori_loop(0, _NB, block, ())


@jax.jit
def kernel(x, tables):
    mesh = plsc.VectorSubcoreMesh(core_axis_name="c", subcore_axis_name="s")
    run = functools.partial(
        pl.kernel,
        mesh=mesh,
        compiler_params=pltpu.CompilerParams(
            use_tc_tiling_on_sc=False, needs_layout_passes=False
        ),
        out_type=jax.ShapeDtypeStruct((BATCH, _OUTW), jnp.float32),
        scratch_types=[
            pltpu.VMEM((_BW, N_FIELDS), jnp.int32),
            pltpu.VMEM((2, _BB), jnp.int32),
            pltpu.VMEM((2, _BB, EMB), jnp.float32),
            pltpu.VMEM((_BB, _OUTW), jnp.float32),
            pltpu.SemaphoreType.DMA,
            pltpu.SemaphoreType.DMA,
        ],
    )(_body)
    return run(x, tables)
